# baseline (speedup 1.0000x reference)
import jax
import jax.numpy as jnp
from jax.experimental import pallas as pl
from jax.experimental.pallas import tpu as pltpu

N = 16
T_PRICE, D_PRICE = 64, 32
T_NEWS, D_NEWS = 2048, 128
H = 64
MIN_NEWS = 10
CHUNK = 256
NCHUNKS = T_NEWS // CHUNK
G3 = 3 * 2 * H


def _pack_input_w(WihT):
    D = WihT.shape[0]
    W = jnp.zeros((D, G3), dtype=WihT.dtype)
    W = W.at[:, 0:H].set(WihT[:, 0:H])
    W = W.at[:, 128:128 + H].set(WihT[:, H:2 * H])
    W = W.at[:, 256:256 + H].set(WihT[:, 2 * H:])
    return W


def _pack_comb_w(Whh0T, Wih1T, Whh1T):
    W = jnp.zeros((2 * H, 2 * G3), dtype=Whh0T.dtype)
    for g in range(3):
        W = W.at[0:H, g * 128 + H:g * 128 + 2 * H].set(Wih1T[:, g * H:(g + 1) * H])
        W = W.at[0:H, G3 + g * 128:G3 + g * 128 + H].set(Whh0T[:, g * H:(g + 1) * H])
        W = W.at[H:2 * H, G3 + g * 128 + H:G3 + g * 128 + 2 * H].set(Whh1T[:, g * H:(g + 1) * H])
    return W


def _pack_bias(b0, b1):
    b = jnp.zeros((1, G3), dtype=b0.dtype)
    for g in range(3):
        b = b.at[0, g * 128:g * 128 + H].set(b0[g * H:(g + 1) * H])
        b = b.at[0, g * 128 + H:g * 128 + 2 * H].set(b1[g * H:(g + 1) * H])
    return b


def _fused_kernel(newsR_ref, maskR_ref, maskf_ref, priceR_ref,
                  win_n_ref, wc_n_ref, bi_n_ref, bhn_n_ref,
                  win_p_ref, wc_p_ref, bi_p_ref, bhn_p_ref,
                  fcw_ref, fcb_ref,
                  out_ref,
                  gi_ref, h_ref, pf_ref):
    i = pl.program_id(0)
    lane = jax.lax.broadcasted_iota(jnp.int32, (N // 2, 2 * H), 1)
    sel_h = lane < H

    def make_step(wc, bhn, t0, pred):
        def half(h, gbase, s):
            comb = jax.lax.dot_general(
                h, wc, (((1,), (0,)), ((), ())),
                preferred_element_type=jnp.float32)
            gsum = gi_ref[pl.ds(gbase, N // 2), :] + comb[:, 0:G3]
            r = jax.nn.sigmoid(gsum[:, 0:128] + comb[:, 384:512])
            z = jax.nn.sigmoid(gsum[:, 128:256] + comb[:, 512:640])
            ng = jnp.tanh(gsum[:, 256:384] + r * (comb[:, 640:768] + bhn))
            hn = (1.0 - z) * ng + z * h
            if not pred:
                return hn
            return jnp.where(sel_h | (t0 + s >= 1), hn, h)

        def step(s, hs):
            ha, hb = hs
            return (half(ha, s * N, s), half(hb, s * N + N // 2, s))
        return step

    def final_step(hs, wc, bi, bhn):
        def fin_half(h):
            comb = jax.lax.dot_general(
                h, wc, (((1,), (0,)), ((), ())),
                preferred_element_type=jnp.float32)
            gsum = comb[:, 0:G3] + bi
            r = jax.nn.sigmoid(gsum[:, 0:128] + comb[:, 384:512])
            z = jax.nn.sigmoid(gsum[:, 128:256] + comb[:, 512:640])
            ng = jnp.tanh(gsum[:, 256:384] + r * (comb[:, 640:768] + bhn))
            hn = (1.0 - z) * ng + z * h
            return hn[:, H:2 * H]
        return jnp.concatenate([fin_half(hs[0]), fin_half(hs[1])], axis=0)

    @pl.when(i == 0)
    def _price_and_init():
        gi_ref[pl.ds(0, T_PRICE * N), :] = jax.lax.dot_general(
            priceR_ref[...], win_p_ref[...], (((1,), (0,)), ((), ())),
            preferred_element_type=jnp.float32) + bi_p_ref[...]
        hz = jnp.zeros((N // 2, 2 * H), dtype=jnp.float32)
        hs = jax.lax.fori_loop(
            0, T_PRICE, make_step(wc_p_ref[...], bhn_p_ref[...], 0, True),
            (hz, hz), unroll=4)
        pf_ref[...] = final_step(hs, wc_p_ref[...], bi_p_ref[...], bhn_p_ref[...])
        h_ref[...] = jnp.zeros((N, 2 * H), dtype=jnp.float32)

    xz = newsR_ref[...] * (1.0 - maskR_ref[...])
    gi_ref[...] = jax.lax.dot_general(
        xz, win_n_ref[...], (((1,), (0,)), ((), ())),
        preferred_element_type=jnp.float32) + bi_n_ref[...]

    def scan_chunk(pred):
        hs = (h_ref[0:N // 2, :], h_ref[N // 2:N, :])
        hs = jax.lax.fori_loop(
            0, CHUNK, make_step(wc_n_ref[...], bhn_n_ref[...], 0, pred),
            hs, unroll=4)
        h_ref[0:N // 2, :] = hs[0]
        h_ref[N // 2:N, :] = hs[1]

    @pl.when(i == 0)
    def _scan_first():
        scan_chunk(True)

    @pl.when(i > 0)
    def _scan_rest():
        scan_chunk(False)

    @pl.when(i == NCHUNKS - 1)
    def _finish():
        news_feat = final_step(
            (h_ref[0:N // 2, :], h_ref[N // 2:N, :]),
            wc_n_ref[...], bi_n_ref[...], bhn_n_ref[...])
        news_len = jnp.sum(1.0 - maskf_ref[...], axis=1, keepdims=True)
        gate = (news_len >= float(MIN_NEWS)).astype(jnp.float32)
        fused = pf_ref[...] + gate * news_feat
        out_ref[...] = jax.lax.dot_general(
            fused, fcw_ref[...], (((1,), (0,)), ((), ())),
            preferred_element_type=jnp.float32) + fcb_ref[...]


def kernel(price, news, mask, Wih0_p, Whh0_p, bih0_p, bhh0_p, Wih1_p, Whh1_p,
           bih1_p, bhh1_p, Wih0_n, Whh0_n, bih0_n, bhh0_n, Wih1_n, Whh1_n,
           bih1_n, bhh1_n, fc_w, fc_b):
    maskf = mask.astype(jnp.float32)
    newsR = jnp.swapaxes(news, 0, 1).reshape(T_NEWS * N, D_NEWS)
    maskR = maskf.T.reshape(T_NEWS * N, 1)
    priceR = jnp.swapaxes(price, 0, 1).reshape(T_PRICE * N, D_PRICE)

    def fold_rz(bi, bh):
        return bi + jnp.concatenate([bh[:2 * H], jnp.zeros((H,), bh.dtype)])

    win_n = _pack_input_w(Wih0_n.T)
    wc_n = _pack_comb_w(Whh0_n.T, Wih1_n.T, Whh1_n.T)
    bi_n = _pack_bias(fold_rz(bih0_n, bhh0_n), fold_rz(bih1_n, bhh1_n))
    bhn_n = jnp.concatenate([bhh0_n[2 * H:], bhh1_n[2 * H:]]).reshape(1, 2 * H)
    win_p = _pack_input_w(Wih0_p.T)
    wc_p = _pack_comb_w(Whh0_p.T, Wih1_p.T, Whh1_p.T)
    bi_p = _pack_bias(fold_rz(bih0_p, bhh0_p), fold_rz(bih1_p, bhh1_p))
    bhn_p = jnp.concatenate([bhh0_p[2 * H:], bhh1_p[2 * H:]]).reshape(1, 2 * H)

    fcw = jnp.zeros((H, 128), dtype=jnp.float32).at[:, 0].set(fc_w[0])
    fcb = jnp.zeros((1, 128), dtype=jnp.float32).at[0, 0].set(fc_b[0])

    full = lambda *shape: pl.BlockSpec(shape, lambda i: tuple(0 for _ in shape))

    out = pl.pallas_call(
        _fused_kernel,
        grid=(NCHUNKS,),
        in_specs=[
            pl.BlockSpec((CHUNK * N, D_NEWS), lambda i: (i, 0)),
            pl.BlockSpec((CHUNK * N, 1), lambda i: (i, 0)),
            full(N, T_NEWS),
            full(T_PRICE * N, D_PRICE),
            full(D_NEWS, G3),
            full(2 * H, 2 * G3),
            full(1, G3),
            full(1, 2 * H),
            full(D_PRICE, G3),
            full(2 * H, 2 * G3),
            full(1, G3),
            full(1, 2 * H),
            full(H, 128),
            full(1, 128),
        ],
        out_specs=pl.BlockSpec((N, 128), lambda i: (0, 0)),
        out_shape=jax.ShapeDtypeStruct((N, 128), jnp.float32),
        scratch_shapes=[
            pltpu.VMEM((CHUNK * N, G3), jnp.float32),
            pltpu.VMEM((N, 2 * H), jnp.float32),
            pltpu.VMEM((N, H), jnp.float32),
        ],
        compiler_params=pltpu.CompilerParams(
            dimension_semantics=("arbitrary",)),
    )(newsR, maskR, maskf, priceR,
      win_n, wc_n, bi_n, bhn_n,
      win_p, wc_p, bi_p, bhn_p,
      fcw, fcb)
    return out[:, :1]


# speedup vs baseline: 1.0598x; 1.0598x over previous
import jax
import jax.numpy as jnp
from jax.experimental import pallas as pl
from jax.experimental.pallas import tpu as pltpu

N = 16
T_PRICE, D_PRICE = 64, 32
T_NEWS, D_NEWS = 2048, 128
H = 64
MIN_NEWS = 10
CHUNK = 256
NCHUNKS = T_NEWS // CHUNK
G3 = 3 * 2 * H


def _pack_input_w(WihT):
    D = WihT.shape[0]
    W = jnp.zeros((D, G3), dtype=WihT.dtype)
    W = W.at[:, 0:H].set(WihT[:, 0:H])
    W = W.at[:, 128:128 + H].set(WihT[:, H:2 * H])
    W = W.at[:, 256:256 + H].set(WihT[:, 2 * H:])
    return W


def _pack_comb_w(Whh0T, Wih1T, Whh1T):
    W = jnp.zeros((2 * H, 2 * G3), dtype=Whh0T.dtype)
    for g in range(3):
        W = W.at[0:H, g * 128 + H:g * 128 + 2 * H].set(Wih1T[:, g * H:(g + 1) * H])
        W = W.at[0:H, G3 + g * 128:G3 + g * 128 + H].set(Whh0T[:, g * H:(g + 1) * H])
        W = W.at[H:2 * H, G3 + g * 128 + H:G3 + g * 128 + 2 * H].set(Whh1T[:, g * H:(g + 1) * H])
    return W


def _pack_bias(b0, b1):
    b = jnp.zeros((1, G3), dtype=b0.dtype)
    for g in range(3):
        b = b.at[0, g * 128:g * 128 + H].set(b0[g * H:(g + 1) * H])
        b = b.at[0, g * 128 + H:g * 128 + 2 * H].set(b1[g * H:(g + 1) * H])
    return b


def _fused_kernel(newsR_ref, maskR_ref, maskf_ref, priceR_ref,
                  win_n_ref, wc_n_ref, bi_n_ref, bhn_n_ref,
                  win_p_ref, wc_p_ref, bi_p_ref, bhn_p_ref,
                  fcw_ref, fcb_ref,
                  out_ref,
                  gi_ref, h_ref, pf_ref):
    i = pl.program_id(0)
    lane = jax.lax.broadcasted_iota(jnp.int32, (N, 2 * H), 1)
    sel_h = lane < H

    def make_step(wc, bhn, t0, pred):
        def step(s, h):
            comb = jax.lax.dot_general(
                h, wc, (((1,), (0,)), ((), ())),
                preferred_element_type=jnp.float32)
            gsum = gi_ref[pl.ds(s * N, N), :] + comb[:, 0:G3]
            r = jax.nn.sigmoid(gsum[:, 0:128] + comb[:, 384:512])
            z = jax.nn.sigmoid(gsum[:, 128:256] + comb[:, 512:640])
            ng = jnp.tanh(gsum[:, 256:384] + r * (comb[:, 640:768] + bhn))
            hn = (1.0 - z) * ng + z * h
            if not pred:
                return hn
            return jnp.where(sel_h | (t0 + s >= 1), hn, h)
        return step

    def final_step(h, wc, bi, bhn):
        comb = jax.lax.dot_general(
            h, wc, (((1,), (0,)), ((), ())),
            preferred_element_type=jnp.float32)
        gsum = comb[:, 0:G3] + bi
        r = jax.nn.sigmoid(gsum[:, 0:128] + comb[:, 384:512])
        z = jax.nn.sigmoid(gsum[:, 128:256] + comb[:, 512:640])
        ng = jnp.tanh(gsum[:, 256:384] + r * (comb[:, 640:768] + bhn))
        hn = (1.0 - z) * ng + z * h
        return hn[:, H:2 * H]

    @pl.when(i == 0)
    def _price_and_init():
        gi_ref[pl.ds(0, T_PRICE * N), :] = jax.lax.dot_general(
            priceR_ref[...], win_p_ref[...], (((1,), (0,)), ((), ())),
            preferred_element_type=jnp.float32) + bi_p_ref[...]
        hz = jnp.zeros((N, 2 * H), dtype=jnp.float32)
        hs = jax.lax.fori_loop(
            0, T_PRICE, make_step(wc_p_ref[...], bhn_p_ref[...], 0, True),
            hz, unroll=4)
        pf_ref[...] = final_step(hs, wc_p_ref[...], bi_p_ref[...], bhn_p_ref[...])
        h_ref[...] = jnp.zeros((N, 2 * H), dtype=jnp.float32)

    xT = jnp.swapaxes(newsR_ref[...], 0, 1).reshape(CHUNK * N, D_NEWS)
    gi_ref[...] = jax.lax.dot_general(
        xT, win_n_ref[...], (((1,), (0,)), ((), ())),
        preferred_element_type=jnp.float32) * (1.0 - maskR_ref[...]) \
        + bi_n_ref[...]

    def scan_chunk(pred):
        h_ref[...] = jax.lax.fori_loop(
            0, CHUNK, make_step(wc_n_ref[...], bhn_n_ref[...], 0, pred),
            h_ref[...], unroll=4)

    @pl.when(i == 0)
    def _scan_first():
        scan_chunk(True)

    @pl.when(i > 0)
    def _scan_rest():
        scan_chunk(False)

    @pl.when(i == NCHUNKS - 1)
    def _finish():
        news_feat = final_step(
            h_ref[...], wc_n_ref[...], bi_n_ref[...], bhn_n_ref[...])
        news_len = jnp.sum(1.0 - maskf_ref[...], axis=1, keepdims=True)
        gate = (news_len >= float(MIN_NEWS)).astype(jnp.float32)
        fused = pf_ref[...] + gate * news_feat
        out_ref[...] = jax.lax.dot_general(
            fused, fcw_ref[...], (((1,), (0,)), ((), ())),
            preferred_element_type=jnp.float32) + fcb_ref[...]


def kernel(price, news, mask, Wih0_p, Whh0_p, bih0_p, bhh0_p, Wih1_p, Whh1_p,
           bih1_p, bhh1_p, Wih0_n, Whh0_n, bih0_n, bhh0_n, Wih1_n, Whh1_n,
           bih1_n, bhh1_n, fc_w, fc_b):
    maskf = mask.astype(jnp.float32)
    maskR = maskf.T.reshape(T_NEWS * N, 1)
    priceR = jnp.swapaxes(price, 0, 1).reshape(T_PRICE * N, D_PRICE)

    def fold_rz(bi, bh):
        return bi + jnp.concatenate([bh[:2 * H], jnp.zeros((H,), bh.dtype)])

    win_n = _pack_input_w(Wih0_n.T)
    wc_n = _pack_comb_w(Whh0_n.T, Wih1_n.T, Whh1_n.T)
    bi_n = _pack_bias(fold_rz(bih0_n, bhh0_n), fold_rz(bih1_n, bhh1_n))
    bhn_n = jnp.concatenate([bhh0_n[2 * H:], bhh1_n[2 * H:]]).reshape(1, 2 * H)
    win_p = _pack_input_w(Wih0_p.T)
    wc_p = _pack_comb_w(Whh0_p.T, Wih1_p.T, Whh1_p.T)
    bi_p = _pack_bias(fold_rz(bih0_p, bhh0_p), fold_rz(bih1_p, bhh1_p))
    bhn_p = jnp.concatenate([bhh0_p[2 * H:], bhh1_p[2 * H:]]).reshape(1, 2 * H)

    fcw = jnp.zeros((H, 128), dtype=jnp.float32).at[:, 0].set(fc_w[0])
    fcb = jnp.zeros((1, 128), dtype=jnp.float32).at[0, 0].set(fc_b[0])

    full = lambda *shape: pl.BlockSpec(shape, lambda i: tuple(0 for _ in shape))

    out = pl.pallas_call(
        _fused_kernel,
        grid=(NCHUNKS,),
        in_specs=[
            pl.BlockSpec((N, CHUNK, D_NEWS), lambda i: (0, i, 0)),
            pl.BlockSpec((CHUNK * N, 1), lambda i: (i, 0)),
            full(N, T_NEWS),
            full(T_PRICE * N, D_PRICE),
            full(D_NEWS, G3),
            full(2 * H, 2 * G3),
            full(1, G3),
            full(1, 2 * H),
            full(D_PRICE, G3),
            full(2 * H, 2 * G3),
            full(1, G3),
            full(1, 2 * H),
            full(H, 128),
            full(1, 128),
        ],
        out_specs=pl.BlockSpec((N, 128), lambda i: (0, 0)),
        out_shape=jax.ShapeDtypeStruct((N, 128), jnp.float32),
        scratch_shapes=[
            pltpu.VMEM((CHUNK * N, G3), jnp.float32),
            pltpu.VMEM((N, 2 * H), jnp.float32),
            pltpu.VMEM((N, H), jnp.float32),
        ],
        compiler_params=pltpu.CompilerParams(
            dimension_semantics=("arbitrary",)),
    )(news, maskR, maskf, priceR,
      win_n, wc_n, bi_n, bhn_n,
      win_p, wc_p, bi_p, bhn_p,
      fcw, fcb)
    return out[:, :1]


# speedup vs baseline: 1.3354x; 1.2600x over previous
import jax
import jax.numpy as jnp
from jax.experimental import pallas as pl
from jax.experimental.pallas import tpu as pltpu

N = 16
T_PRICE, D_PRICE = 64, 32
T_NEWS, D_NEWS = 2048, 128
H = 64
MIN_NEWS = 10
CHUNK = 256
NCHUNKS = T_NEWS // CHUNK
G3 = 3 * 2 * H


def _fused_kernel(news_ref, maskR_ref, maskf_ref, price_ref,
                  wih0n_ref, whh0n_ref, wih1n_ref, whh1n_ref,
                  wih0p_ref, whh0p_ref, wih1p_ref, whh1p_ref,
                  bstack_ref, fcw_ref, fcb_ref,
                  out_ref,
                  gi_ref, h_ref, pf_ref,
                  win_n_s, wc_n_s, bi_n_s, bhn_n_s,
                  win_p_s, wc_p_s, bi_p_s, bhn_p_s):
    i = pl.program_id(0)
    lane = jax.lax.broadcasted_iota(jnp.int32, (N, 2 * H), 1)
    sel_h = lane < H

    def make_step(wc, bhn, t0, pred):
        def step(s, h):
            comb = jax.lax.dot_general(
                h, wc, (((1,), (0,)), ((), ())),
                preferred_element_type=jnp.float32)
            gsum = gi_ref[pl.ds(s * N, N), :] + comb[:, 0:G3]
            r = jax.nn.sigmoid(gsum[:, 0:128] + comb[:, 384:512])
            z = jax.nn.sigmoid(gsum[:, 128:256] + comb[:, 512:640])
            ng = jnp.tanh(gsum[:, 256:384] + r * (comb[:, 640:768] + bhn))
            hn = (1.0 - z) * ng + z * h
            if not pred:
                return hn
            return jnp.where(sel_h | (t0 + s >= 1), hn, h)
        return step

    def final_step(h, wc, bi, bhn):
        comb = jax.lax.dot_general(
            h, wc, (((1,), (0,)), ((), ())),
            preferred_element_type=jnp.float32)
        gsum = comb[:, 0:G3] + bi
        r = jax.nn.sigmoid(gsum[:, 0:128] + comb[:, 384:512])
        z = jax.nn.sigmoid(gsum[:, 128:256] + comb[:, 512:640])
        ng = jnp.tanh(gsum[:, 256:384] + r * (comb[:, 640:768] + bhn))
        hn = (1.0 - z) * ng + z * h
        return hn[:, H:2 * H]

    @pl.when(i == 0)
    def _pack_and_price():
        def pack(win_s, wc_s, wih0, whh0, wih1, whh1):
            win_s[...] = jnp.zeros_like(win_s)
            wc_s[...] = jnp.zeros_like(wc_s)
            for g in range(3):
                gl = slice(g * 64, g * 64 + 64)
                win_s[:, g * 128:g * 128 + 64] = wih0[gl, :].T
                wc_s[0:64, g * 128 + 64:g * 128 + 128] = wih1[gl, :].T
                wc_s[0:64, 384 + g * 128:384 + g * 128 + 64] = whh0[gl, :].T
                wc_s[64:128, 384 + g * 128 + 64:384 + g * 128 + 128] = \
                    whh1[gl, :].T

        pack(win_n_s, wc_n_s, wih0n_ref[...], whh0n_ref[...],
             wih1n_ref[...], whh1n_ref[...])
        pack(win_p_s, wc_p_s, wih0p_ref[...], whh0p_ref[...],
             wih1p_ref[...], whh1p_ref[...])

        b = bstack_ref[...]
        for (bi_s, bhn_s, o) in ((bi_n_s, bhn_n_s, 0), (bi_p_s, bhn_p_s, 4)):
            for g in range(3):
                gl = slice(g * 64, g * 64 + 64)
                l0 = b[o:o + 1, gl]
                l1 = b[o + 2:o + 3, gl]
                if g < 2:
                    l0 = l0 + b[o + 1:o + 2, gl]
                    l1 = l1 + b[o + 3:o + 4, gl]
                bi_s[:, g * 128:g * 128 + 64] = l0
                bi_s[:, g * 128 + 64:g * 128 + 128] = l1
            bhn_s[:, 0:64] = b[o + 1:o + 2, 128:192]
            bhn_s[:, 64:128] = b[o + 3:o + 4, 128:192]

        pT = jnp.swapaxes(price_ref[...], 0, 1).reshape(T_PRICE * N, D_PRICE)
        gi_ref[pl.ds(0, T_PRICE * N), :] = jax.lax.dot_general(
            pT, win_p_s[...], (((1,), (0,)), ((), ())),
            preferred_element_type=jnp.float32) + bi_p_s[...]
        hz = jnp.zeros((N, 2 * H), dtype=jnp.float32)
        hp = jax.lax.fori_loop(
            0, T_PRICE, make_step(wc_p_s[...], bhn_p_s[...], 0, True),
            hz, unroll=4)
        pf_ref[...] = final_step(hp, wc_p_s[...], bi_p_s[...], bhn_p_s[...])
        h_ref[...] = jnp.zeros((N, 2 * H), dtype=jnp.float32)

    xT = jnp.swapaxes(news_ref[...], 0, 1).reshape(CHUNK * N, D_NEWS)
    gi_ref[...] = jax.lax.dot_general(
        xT, win_n_s[...], (((1,), (0,)), ((), ())),
        preferred_element_type=jnp.float32) * (1.0 - maskR_ref[...]) \
        + bi_n_s[...]

    def scan_chunk(pred):
        h_ref[...] = jax.lax.fori_loop(
            0, CHUNK, make_step(wc_n_s[...], bhn_n_s[...], 0, pred),
            h_ref[...], unroll=4)

    @pl.when(i == 0)
    def _scan_first():
        scan_chunk(True)

    @pl.when(i > 0)
    def _scan_rest():
        scan_chunk(False)

    @pl.when(i == NCHUNKS - 1)
    def _finish():
        news_feat = final_step(
            h_ref[...], wc_n_s[...], bi_n_s[...], bhn_n_s[...])
        news_len = jnp.sum(1.0 - maskf_ref[...], axis=1, keepdims=True)
        gate = (news_len >= float(MIN_NEWS)).astype(jnp.float32)
        fused = pf_ref[...] + gate * news_feat
        out_ref[:, 0:1] = (jnp.sum(fused * fcw_ref[...], axis=1,
                                   keepdims=True) + fcb_ref[...])


def kernel(price, news, mask, Wih0_p, Whh0_p, bih0_p, bhh0_p, Wih1_p, Whh1_p,
           bih1_p, bhh1_p, Wih0_n, Whh0_n, bih0_n, bhh0_n, Wih1_n, Whh1_n,
           bih1_n, bhh1_n, fc_w, fc_b):
    maskf = mask.astype(jnp.float32)
    maskR = maskf.T.reshape(T_NEWS * N, 1)
    bstack = jnp.stack([bih0_n, bhh0_n, bih1_n, bhh1_n,
                        bih0_p, bhh0_p, bih1_p, bhh1_p])
    fcb = fc_b.reshape(1, 1)

    full = lambda *shape: pl.BlockSpec(shape, lambda i: tuple(0 for _ in shape))

    out = pl.pallas_call(
        _fused_kernel,
        grid=(NCHUNKS,),
        in_specs=[
            pl.BlockSpec((N, CHUNK, D_NEWS), lambda i: (0, i, 0)),
            pl.BlockSpec((CHUNK * N, 1), lambda i: (i, 0)),
            full(N, T_NEWS),
            full(N, T_PRICE, D_PRICE),
            full(3 * H, D_NEWS),
            full(3 * H, H),
            full(3 * H, H),
            full(3 * H, H),
            full(3 * H, D_PRICE),
            full(3 * H, H),
            full(3 * H, H),
            full(3 * H, H),
            full(8, 3 * H),
            full(1, H),
            full(1, 1),
        ],
        out_specs=pl.BlockSpec((N, 128), lambda i: (0, 0)),
        out_shape=jax.ShapeDtypeStruct((N, 128), jnp.float32),
        scratch_shapes=[
            pltpu.VMEM((CHUNK * N, G3), jnp.float32),
            pltpu.VMEM((N, 2 * H), jnp.float32),
            pltpu.VMEM((N, H), jnp.float32),
            pltpu.VMEM((D_NEWS, G3), jnp.float32),
            pltpu.VMEM((2 * H, 2 * G3), jnp.float32),
            pltpu.VMEM((1, G3), jnp.float32),
            pltpu.VMEM((1, 2 * H), jnp.float32),
            pltpu.VMEM((D_PRICE, G3), jnp.float32),
            pltpu.VMEM((2 * H, 2 * G3), jnp.float32),
            pltpu.VMEM((1, G3), jnp.float32),
            pltpu.VMEM((1, 2 * H), jnp.float32),
        ],
        compiler_params=pltpu.CompilerParams(
            dimension_semantics=("arbitrary",)),
    )(news, maskR, maskf, price,
      Wih0_n, Whh0_n, Wih1_n, Whh1_n,
      Wih0_p, Whh0_p, Wih1_p, Whh1_p,
      bstack, fc_w, fcb)
    return out[:, :1]


# speedup vs baseline: 1.3862x; 1.0380x over previous
import jax
import jax.numpy as jnp
from jax.experimental import pallas as pl
from jax.experimental.pallas import tpu as pltpu

N = 16
T_PRICE, D_PRICE = 64, 32
T_NEWS, D_NEWS = 2048, 128
H = 64
MIN_NEWS = 10
CHUNK = 256
NCHUNKS = T_NEWS // CHUNK
G3 = 3 * 2 * H


def _fused_kernel(news_ref, maskR_ref, maskf_ref, price_ref,
                  wih0n_ref, whh0n_ref, wih1n_ref, whh1n_ref,
                  wih0p_ref, whh0p_ref, wih1p_ref, whh1p_ref,
                  bstack_ref, fcw_ref, fcb_ref,
                  out_ref,
                  gi_ref, h_ref, pf_ref,
                  win_n_s, wc_n_s, bi_n_s, bhn_n_s,
                  win_p_s, wc_p_s, bi_p_s, bhn_p_s):
    i = pl.program_id(0)
    lane = jax.lax.broadcasted_iota(jnp.int32, (N, 2 * H), 1)
    sel_h = lane < H

    def gru_math(gp, c1, c2, h, bhn):
        r = jax.nn.sigmoid(gp[:, 0:128] + c1[:, 0:128] + c1[:, 128:256])
        z = jax.nn.sigmoid(gp[:, 128:256] + c2[:, 0:128] + c2[:, 128:256])
        ng = jnp.tanh(gp[:, 256:384] + c2[:, 256:384]
                      + r * (c2[:, 384:512] + bhn))
        return (1.0 - z) * ng + z * h

    def two_dots(h, wc):
        c1 = jax.lax.dot_general(
            h, wc[:, 0:256], (((1,), (0,)), ((), ())),
            preferred_element_type=jnp.float32)
        c2 = jax.lax.dot_general(
            h, wc[:, 256:768], (((1,), (0,)), ((), ())),
            preferred_element_type=jnp.float32)
        return c1, c2

    def make_step(wc, bhn, t0, pred):
        def step(s, h):
            c1, c2 = two_dots(h, wc)
            gp = gi_ref[pl.ds(s * N, N), :]
            hn = gru_math(gp, c1, c2, h, bhn)
            if not pred:
                return hn
            return jnp.where(sel_h | (t0 + s >= 1), hn, h)
        return step

    def final_step(h, wc, bi, bhn):
        c1, c2 = two_dots(h, wc)
        hn = gru_math(bi, c1, c2, h, bhn)
        return hn[:, H:2 * H]

    @pl.when(i == 0)
    def _pack_and_price():
        def pack(win_s, wc_s, wih0, whh0, wih1, whh1):
            win_s[...] = jnp.zeros_like(win_s)
            wc_s[...] = jnp.zeros_like(wc_s)
            for g in range(3):
                gl = slice(g * 64, g * 64 + 64)
                base = 256 * g
                win_s[:, g * 128:g * 128 + 64] = wih0[gl, :].T
                wc_s[0:64, base + 64:base + 128] = wih1[gl, :].T
                wc_s[0:64, base + 128:base + 192] = whh0[gl, :].T
                wc_s[64:128, base + 192:base + 256] = whh1[gl, :].T

        pack(win_n_s, wc_n_s, wih0n_ref[...], whh0n_ref[...],
             wih1n_ref[...], whh1n_ref[...])
        pack(win_p_s, wc_p_s, wih0p_ref[...], whh0p_ref[...],
             wih1p_ref[...], whh1p_ref[...])

        b = bstack_ref[...]
        for (bi_s, bhn_s, o) in ((bi_n_s, bhn_n_s, 0), (bi_p_s, bhn_p_s, 4)):
            for g in range(3):
                gl = slice(g * 64, g * 64 + 64)
                l0 = b[o:o + 1, gl]
                l1 = b[o + 2:o + 3, gl]
                if g < 2:
                    l0 = l0 + b[o + 1:o + 2, gl]
                    l1 = l1 + b[o + 3:o + 4, gl]
                bi_s[:, g * 128:g * 128 + 64] = l0
                bi_s[:, g * 128 + 64:g * 128 + 128] = l1
            bhn_s[:, 0:64] = b[o + 1:o + 2, 128:192]
            bhn_s[:, 64:128] = b[o + 3:o + 4, 128:192]

        pT = jnp.swapaxes(price_ref[...], 0, 1).reshape(T_PRICE * N, D_PRICE)
        gi_ref[pl.ds(0, T_PRICE * N), :] = jax.lax.dot_general(
            pT, win_p_s[...], (((1,), (0,)), ((), ())),
            preferred_element_type=jnp.float32) + bi_p_s[...]
        hz = jnp.zeros((N, 2 * H), dtype=jnp.float32)
        hp = jax.lax.fori_loop(
            0, T_PRICE, make_step(wc_p_s[...], bhn_p_s[...], 0, True),
            hz, unroll=4)
        pf_ref[...] = final_step(hp, wc_p_s[...], bi_p_s[...], bhn_p_s[...])
        h_ref[...] = jnp.zeros((N, 2 * H), dtype=jnp.float32)

    xT = jnp.swapaxes(news_ref[...], 0, 1).reshape(CHUNK * N, D_NEWS)
    gi_ref[...] = jax.lax.dot_general(
        xT, win_n_s[...], (((1,), (0,)), ((), ())),
        preferred_element_type=jnp.float32) * (1.0 - maskR_ref[...]) \
        + bi_n_s[...]

    def scan_chunk(pred):
        h_ref[...] = jax.lax.fori_loop(
            0, CHUNK, make_step(wc_n_s[...], bhn_n_s[...], 0, pred),
            h_ref[...], unroll=4)

    @pl.when(i == 0)
    def _scan_first():
        scan_chunk(True)

    @pl.when(i > 0)
    def _scan_rest():
        scan_chunk(False)

    @pl.when(i == NCHUNKS - 1)
    def _finish():
        news_feat = final_step(
            h_ref[...], wc_n_s[...], bi_n_s[...], bhn_n_s[...])
        news_len = jnp.sum(1.0 - maskf_ref[...], axis=1, keepdims=True)
        gate = (news_len >= float(MIN_NEWS)).astype(jnp.float32)
        fused = pf_ref[...] + gate * news_feat
        out_ref[:, 0:1] = (jnp.sum(fused * fcw_ref[...], axis=1,
                                   keepdims=True) + fcb_ref[...])


def kernel(price, news, mask, Wih0_p, Whh0_p, bih0_p, bhh0_p, Wih1_p, Whh1_p,
           bih1_p, bhh1_p, Wih0_n, Whh0_n, bih0_n, bhh0_n, Wih1_n, Whh1_n,
           bih1_n, bhh1_n, fc_w, fc_b):
    maskf = mask.astype(jnp.float32)
    maskR = maskf.T.reshape(T_NEWS * N, 1)
    bstack = jnp.stack([bih0_n, bhh0_n, bih1_n, bhh1_n,
                        bih0_p, bhh0_p, bih1_p, bhh1_p])
    fcb = fc_b.reshape(1, 1)

    full = lambda *shape: pl.BlockSpec(shape, lambda i: tuple(0 for _ in shape))

    out = pl.pallas_call(
        _fused_kernel,
        grid=(NCHUNKS,),
        in_specs=[
            pl.BlockSpec((N, CHUNK, D_NEWS), lambda i: (0, i, 0)),
            pl.BlockSpec((CHUNK * N, 1), lambda i: (i, 0)),
            full(N, T_NEWS),
            full(N, T_PRICE, D_PRICE),
            full(3 * H, D_NEWS),
            full(3 * H, H),
            full(3 * H, H),
            full(3 * H, H),
            full(3 * H, D_PRICE),
            full(3 * H, H),
            full(3 * H, H),
            full(3 * H, H),
            full(8, 3 * H),
            full(1, H),
            full(1, 1),
        ],
        out_specs=pl.BlockSpec((N, 128), lambda i: (0, 0)),
        out_shape=jax.ShapeDtypeStruct((N, 128), jnp.float32),
        scratch_shapes=[
            pltpu.VMEM((CHUNK * N, G3), jnp.float32),
            pltpu.VMEM((N, 2 * H), jnp.float32),
            pltpu.VMEM((N, H), jnp.float32),
            pltpu.VMEM((D_NEWS, G3), jnp.float32),
            pltpu.VMEM((2 * H, 2 * G3), jnp.float32),
            pltpu.VMEM((1, G3), jnp.float32),
            pltpu.VMEM((1, 2 * H), jnp.float32),
            pltpu.VMEM((D_PRICE, G3), jnp.float32),
            pltpu.VMEM((2 * H, 2 * G3), jnp.float32),
            pltpu.VMEM((1, G3), jnp.float32),
            pltpu.VMEM((1, 2 * H), jnp.float32),
        ],
        compiler_params=pltpu.CompilerParams(
            dimension_semantics=("arbitrary",)),
    )(news, maskR, maskf, price,
      Wih0_n, Whh0_n, Wih1_n, Whh1_n,
      Wih0_p, Whh0_p, Wih1_p, Whh1_p,
      bstack, fc_w, fcb)
    return out[:, :1]


# speedup vs baseline: 1.4097x; 1.0169x over previous
import jax
import jax.numpy as jnp
from jax.experimental import pallas as pl
from jax.experimental.pallas import tpu as pltpu

N = 16
T_PRICE, D_PRICE = 64, 32
T_NEWS, D_NEWS = 2048, 128
H = 64
MIN_NEWS = 10
CHUNK = 256
NCHUNKS = T_NEWS // CHUNK
G3 = 3 * 2 * H


def _fused_kernel(news_ref, maskR_ref, maskf_ref, price_ref,
                  wih0n_ref, whh0n_ref, wih1n_ref, whh1n_ref,
                  wih0p_ref, whh0p_ref, wih1p_ref, whh1p_ref,
                  bstack_ref, fcw_ref, fcb_ref,
                  out_ref,
                  gi_ref, h_ref, pf_ref,
                  win_n_s, wc_n_s, bi_n_s, bhn_n_s,
                  win_p_s, wc_p_s, bi_p_s, bhn_p_s):
    i = pl.program_id(0)
    lane = jax.lax.broadcasted_iota(jnp.int32, (N, 2 * H), 1)
    sel_h = lane < H

    def gru_math(gp, c1, c2, h, bhn):
        r = jax.nn.sigmoid(gp[:, 0:128] + c1[:, 0:128] + c1[:, 128:256])
        z = jax.nn.sigmoid(gp[:, 128:256] + c2[:, 0:128] + c2[:, 128:256])
        ng = jnp.tanh(gp[:, 256:384] + c2[:, 256:384]
                      + r * (c2[:, 384:512] + bhn))
        return (1.0 - z) * ng + z * h

    def two_dots(h, wc):
        c1 = jax.lax.dot_general(
            h, wc[:, 0:256], (((1,), (0,)), ((), ())),
            preferred_element_type=jnp.float32)
        c2 = jax.lax.dot_general(
            h, wc[:, 256:768], (((1,), (0,)), ((), ())),
            preferred_element_type=jnp.float32)
        return c1, c2

    def make_step(wc, bhn, t0, pred):
        def step(s, h):
            c1, c2 = two_dots(h, wc)
            gp = gi_ref[pl.ds(s * N, N), :]
            hn = gru_math(gp, c1, c2, h, bhn)
            if not pred:
                return hn
            return jnp.where(sel_h | (t0 + s >= 1), hn, h)
        return step

    def final_step(h, wc, bi, bhn):
        c1, c2 = two_dots(h, wc)
        hn = gru_math(bi, c1, c2, h, bhn)
        return hn[:, H:2 * H]

    @pl.when(i == 0)
    def _pack_and_price():
        def pack(win_s, wc_s, wih0, whh0, wih1, whh1):
            win_s[...] = jnp.zeros_like(win_s)
            wc_s[...] = jnp.zeros_like(wc_s)
            for g in range(3):
                gl = slice(g * 64, g * 64 + 64)
                base = 256 * g
                win_s[:, g * 128:g * 128 + 64] = wih0[gl, :].T
                wc_s[0:64, base + 64:base + 128] = wih1[gl, :].T
                wc_s[0:64, base + 128:base + 192] = whh0[gl, :].T
                wc_s[64:128, base + 192:base + 256] = whh1[gl, :].T

        pack(win_n_s, wc_n_s, wih0n_ref[...], whh0n_ref[...],
             wih1n_ref[...], whh1n_ref[...])
        pack(win_p_s, wc_p_s, wih0p_ref[...], whh0p_ref[...],
             wih1p_ref[...], whh1p_ref[...])

        b = bstack_ref[...]
        for (bi_s, bhn_s, o) in ((bi_n_s, bhn_n_s, 0), (bi_p_s, bhn_p_s, 4)):
            for g in range(3):
                gl = slice(g * 64, g * 64 + 64)
                l0 = b[o:o + 1, gl]
                l1 = b[o + 2:o + 3, gl]
                if g < 2:
                    l0 = l0 + b[o + 1:o + 2, gl]
                    l1 = l1 + b[o + 3:o + 4, gl]
                bi_s[:, g * 128:g * 128 + 64] = l0
                bi_s[:, g * 128 + 64:g * 128 + 128] = l1
            bhn_s[:, 0:64] = b[o + 1:o + 2, 128:192]
            bhn_s[:, 64:128] = b[o + 3:o + 4, 128:192]

        pT = jnp.swapaxes(price_ref[...], 0, 1).reshape(T_PRICE * N, D_PRICE)
        gi_ref[pl.ds(0, T_PRICE * N), :] = jax.lax.dot_general(
            pT, win_p_s[...], (((1,), (0,)), ((), ())),
            preferred_element_type=jnp.float32) + bi_p_s[...]
        hz = jnp.zeros((N, 2 * H), dtype=jnp.float32)
        hp = jax.lax.fori_loop(
            0, T_PRICE, make_step(wc_p_s[...], bhn_p_s[...], 0, True),
            hz, unroll=8)
        pf_ref[...] = final_step(hp, wc_p_s[...], bi_p_s[...], bhn_p_s[...])
        h_ref[...] = jnp.zeros((N, 2 * H), dtype=jnp.float32)

    xT = jnp.swapaxes(news_ref[...], 0, 1).reshape(CHUNK * N, D_NEWS)
    gi_ref[...] = jax.lax.dot_general(
        xT, win_n_s[...], (((1,), (0,)), ((), ())),
        preferred_element_type=jnp.float32) * (1.0 - maskR_ref[...]) \
        + bi_n_s[...]

    def scan_chunk(pred):
        h_ref[...] = jax.lax.fori_loop(
            0, CHUNK, make_step(wc_n_s[...], bhn_n_s[...], 0, pred),
            h_ref[...], unroll=8)

    @pl.when(i == 0)
    def _scan_first():
        scan_chunk(True)

    @pl.when(i > 0)
    def _scan_rest():
        scan_chunk(False)

    @pl.when(i == NCHUNKS - 1)
    def _finish():
        news_feat = final_step(
            h_ref[...], wc_n_s[...], bi_n_s[...], bhn_n_s[...])
        news_len = jnp.sum(1.0 - maskf_ref[...], axis=1, keepdims=True)
        gate = (news_len >= float(MIN_NEWS)).astype(jnp.float32)
        fused = pf_ref[...] + gate * news_feat
        out_ref[:, 0:1] = (jnp.sum(fused * fcw_ref[...], axis=1,
                                   keepdims=True) + fcb_ref[...])


def kernel(price, news, mask, Wih0_p, Whh0_p, bih0_p, bhh0_p, Wih1_p, Whh1_p,
           bih1_p, bhh1_p, Wih0_n, Whh0_n, bih0_n, bhh0_n, Wih1_n, Whh1_n,
           bih1_n, bhh1_n, fc_w, fc_b):
    maskf = mask.astype(jnp.float32)
    maskR = maskf.T.reshape(T_NEWS * N, 1)
    bstack = jnp.stack([bih0_n, bhh0_n, bih1_n, bhh1_n,
                        bih0_p, bhh0_p, bih1_p, bhh1_p])
    fcb = fc_b.reshape(1, 1)

    full = lambda *shape: pl.BlockSpec(shape, lambda i: tuple(0 for _ in shape))

    out = pl.pallas_call(
        _fused_kernel,
        grid=(NCHUNKS,),
        in_specs=[
            pl.BlockSpec((N, CHUNK, D_NEWS), lambda i: (0, i, 0)),
            pl.BlockSpec((CHUNK * N, 1), lambda i: (i, 0)),
            full(N, T_NEWS),
            full(N, T_PRICE, D_PRICE),
            full(3 * H, D_NEWS),
            full(3 * H, H),
            full(3 * H, H),
            full(3 * H, H),
            full(3 * H, D_PRICE),
            full(3 * H, H),
            full(3 * H, H),
            full(3 * H, H),
            full(8, 3 * H),
            full(1, H),
            full(1, 1),
        ],
        out_specs=pl.BlockSpec((N, 128), lambda i: (0, 0)),
        out_shape=jax.ShapeDtypeStruct((N, 128), jnp.float32),
        scratch_shapes=[
            pltpu.VMEM((CHUNK * N, G3), jnp.float32),
            pltpu.VMEM((N, 2 * H), jnp.float32),
            pltpu.VMEM((N, H), jnp.float32),
            pltpu.VMEM((D_NEWS, G3), jnp.float32),
            pltpu.VMEM((2 * H, 2 * G3), jnp.float32),
            pltpu.VMEM((1, G3), jnp.float32),
            pltpu.VMEM((1, 2 * H), jnp.float32),
            pltpu.VMEM((D_PRICE, G3), jnp.float32),
            pltpu.VMEM((2 * H, 2 * G3), jnp.float32),
            pltpu.VMEM((1, G3), jnp.float32),
            pltpu.VMEM((1, 2 * H), jnp.float32),
        ],
        compiler_params=pltpu.CompilerParams(
            dimension_semantics=("arbitrary",)),
    )(news, maskR, maskf, price,
      Wih0_n, Whh0_n, Wih1_n, Whh1_n,
      Wih0_p, Whh0_p, Wih1_p, Whh1_p,
      bstack, fc_w, fcb)
    return out[:, :1]
